# baseline (speedup 1.0000x reference)
import functools
import math

import jax
import jax.numpy as jnp
from jax import lax
from jax.experimental import pallas as pl
from jax.experimental.pallas import tpu as pltpu
from jax.experimental.pallas import tpu_sc as plsc

D_MODEL = 64
SCALE = math.sqrt(D_MODEL)

NUM_CORES = 2
NUM_SUBCORES = 16
NUM_WORKERS = NUM_CORES * NUM_SUBCORES

IDX_MINOR = 128
CHUNK = 256
K = CHUNK // IDX_MINOR
NBUF = 2


def _emb_body(x_hbm, lut_hbm, out_hbm, idx_all,
              in0, in1, out0, out1, sg0, sg1, sw0, sw1,
              *, rows_per_worker):
    wid = lax.axis_index("s") * NUM_CORES + lax.axis_index("c")
    base = wid * rows_per_worker
    num_chunks = rows_per_worker // CHUNK
    rounds = num_chunks // NBUF

    ins = (in0, in1)
    outs = (out0, out1)
    sgs = (sg0, sg1)
    sws = (sw0, sw1)

    pltpu.sync_copy(x_hbm.at[pl.ds(base, rows_per_worker)], idx_all)

    def fire_gather(i, b):
        for j in range(K):
            pltpu.async_copy(
                lut_hbm.at[idx_all.at[pl.ds(i * CHUNK + j * IDX_MINOR, IDX_MINOR)]],
                ins[b].at[pl.ds(j * IDX_MINOR, IDX_MINOR)],
                sgs[b],
            )

    def wait_gather(i, b):
        for j in range(K):
            pltpu.make_async_copy(
                lut_hbm.at[idx_all.at[pl.ds(i * CHUNK + j * IDX_MINOR, IDX_MINOR)]],
                ins[b].at[pl.ds(j * IDX_MINOR, IDX_MINOR)],
                sgs[b],
            ).wait()

    def fire_write(i, b):
        pltpu.async_copy(outs[b], out_hbm.at[pl.ds(base + i * CHUNK, CHUNK)], sws[b])

    def wait_write(i, b):
        pltpu.make_async_copy(
            outs[b], out_hbm.at[pl.ds(base + i * CHUNK, CHUNK)], sws[b]
        ).wait()

    def scale(b):
        src = ins[b]
        dst = outs[b]

        def row(r, c):
            for v in range(D_MODEL // 16):
                sl = pl.ds(v * 16, 16)
                dst[r, sl] = src[r, sl] * SCALE
            return c

        lax.fori_loop(0, CHUNK, row, 0, unroll=4)

    for b in range(NBUF):
        fire_gather(b, b)
    for b in range(NBUF):
        wait_gather(b, b)
        scale(b)
        fire_write(b, b)
        fire_gather(b + NBUF, b)

    def round_body(g, c):
        for b in range(NBUF):
            i = g * NBUF + b
            wait_gather(i, b)
            wait_write(i - NBUF, b)
            scale(b)
            fire_write(i, b)
            fire_gather(i + NBUF, b)
        return c

    lax.fori_loop(1, rounds - 1, round_body, 0)

    for b in range(NBUF):
        i = num_chunks - NBUF + b
        wait_gather(i, b)
        wait_write(i - NBUF, b)
        scale(b)
        fire_write(i, b)
    for b in range(NBUF):
        wait_write(num_chunks - NBUF + b, b)


def kernel(x, lut):
    b, s = x.shape
    n = b * s
    assert n % (NUM_WORKERS * CHUNK * NBUF) == 0
    rows_per_worker = n // NUM_WORKERS
    x_flat = x.reshape(n)

    mesh = plsc.VectorSubcoreMesh(core_axis_name="c", subcore_axis_name="s")
    run = pl.kernel(
        functools.partial(_emb_body, rows_per_worker=rows_per_worker),
        out_type=jax.ShapeDtypeStruct((n, D_MODEL), jnp.float32),
        mesh=mesh,
        scratch_types=[
            pltpu.VMEM((rows_per_worker,), jnp.int32),
            pltpu.VMEM((CHUNK, D_MODEL), jnp.float32),
            pltpu.VMEM((CHUNK, D_MODEL), jnp.float32),
            pltpu.VMEM((CHUNK, D_MODEL), jnp.float32),
            pltpu.VMEM((CHUNK, D_MODEL), jnp.float32),
            pltpu.SemaphoreType.DMA,
            pltpu.SemaphoreType.DMA,
            pltpu.SemaphoreType.DMA,
            pltpu.SemaphoreType.DMA,
        ],
        compiler_params=pltpu.CompilerParams(use_tc_tiling_on_sc=False),
    )
    out = run(x_flat, lut)
    return out.reshape(b, s, D_MODEL)
